# baseline (speedup 1.0000x reference)
import functools

import jax
import jax.numpy as jnp
from jax.experimental import pallas as pl
from jax.experimental.pallas import tpu as pltpu

_R = 8192
_S = 64
_C = 32
_GAIN = 1.0


def _softplus(x):
    return jnp.maximum(x, 0.0) + jnp.log1p(jnp.exp(-jnp.abs(x)))


def _tc_body(x_ref, enc_ref, nf_ref, wb_ref, w1t_ref, b0_ref, b1_ref,
             wop_ref, bop_ref, wct_ref, bc_ref, bg_ref,
             out_ref, nlt_ref, awhc_ref, aw_ref, awt_ref):
    s = pl.program_id(0)

    @pl.when(s == 0)
    def _init():
        nlt_ref[...] = jnp.zeros_like(nlt_ref)
        awhc_ref[...] = jnp.zeros_like(awhc_ref)
        aw_ref[...] = jnp.zeros_like(aw_ref)
        awt_ref[...] = jnp.zeros_like(awt_ref)

    x = x_ref[...]
    z = jax.lax.dot_general(wb_ref[...], x, (((1,), (1,)), ((), ())),
                            preferred_element_type=jnp.float32)
    h = _softplus(z[0:_C, :] + b0_ref[...])
    h = _softplus(jnp.dot(w1t_ref[...], h, preferred_element_type=jnp.float32)
                  + b1_ref[...])
    sig_pre = jnp.sum(h * wop_ref[...], axis=0, keepdims=True) + bop_ref[...]
    sigma = _softplus(sig_pre)

    near = nf_ref[0:1, :]
    far = nf_ref[1:2, :]
    sf = (s.astype(jnp.float32) + 0.5) * (1.0 / _S)
    t = near + (far - near) * sf
    delta = (far - near) * (1.0 / _S)
    sd = sigma * delta

    nlt0 = nlt_ref[...]
    wgt = jnp.exp(-nlt0) * (1.0 - jnp.exp(-sd))
    nlt_ref[...] = nlt0 + sd

    hc = _softplus(h + z[_C:2 * _C, :] + enc_ref[...])
    awhc_ref[...] += wgt * hc
    aw_ref[...] += wgt
    awt_ref[...] += wgt * t

    @pl.when(s == _S - 1)
    def _fin():
        mask = 1.0 - jnp.exp(-nlt_ref[...])
        cf = (jnp.dot(wct_ref[...], awhc_ref[...],
                      preferred_element_type=jnp.float32)
              + bc_ref[...] * aw_ref[...])
        fr = cf[0:3, :] + (1.0 - mask) * bg_ref[...]
        out_ref[...] = jnp.concatenate(
            [fr, mask, awt_ref[...], jnp.zeros((3, _R), jnp.float32)], axis=0)


def _render(x, enc_t, nf, wb, w1t, b0, b1, wop, bop, wct, bc, bg,
            interpret=False):
    grid = (_S,)
    full = lambda shape: pl.BlockSpec(shape, lambda s: (0, 0))
    out = pl.pallas_call(
        _tc_body,
        grid=grid,
        in_specs=[
            pl.BlockSpec((_R, 64), lambda s: (s, 0)),
            full((_C, _R)), full((2, _R)), full((64, 64)), full((_C, _C)),
            full((_C, 1)), full((_C, 1)), full((_C, 1)), full((1, 1)),
            full((16, _C)), full((16, 1)), full((3, 1)),
        ],
        out_specs=full((8, _R)),
        out_shape=jax.ShapeDtypeStruct((8, _R), jnp.float32),
        scratch_shapes=[
            pltpu.VMEM((1, _R), jnp.float32),
            pltpu.VMEM((_C, _R), jnp.float32),
            pltpu.VMEM((1, _R), jnp.float32),
            pltpu.VMEM((1, _R), jnp.float32),
        ],
        compiler_params=pltpu.CompilerParams(
            dimension_semantics=("arbitrary",)),
        interpret=interpret,
    )(x, enc_t, nf, wb, w1t, b0, b1, wop, bop, wct, bc, bg)
    return out


def _bilerp_sr(plane, u, v):
    c, h, w = plane.shape
    x = jnp.clip((u + 1.0) * 0.5 * (w - 1), 0.0, w - 1.0)
    y = jnp.clip((v + 1.0) * 0.5 * (h - 1), 0.0, h - 1.0)
    x0 = jnp.floor(x).astype(jnp.int32)
    y0 = jnp.floor(y).astype(jnp.int32)
    x1 = jnp.minimum(x0 + 1, w - 1)
    y1 = jnp.minimum(y0 + 1, h - 1)
    tx = (x - x0)[..., None]
    ty = (y - y0)[..., None]
    p = jnp.transpose(plane, (1, 2, 0))
    v00 = p[y0, x0]
    v10 = p[y0, x1]
    v01 = p[y1, x0]
    v11 = p[y1, x1]
    return (1.0 - ty) * ((1.0 - tx) * v00 + tx * v10) + ty * ((1.0 - tx) * v01 + tx * v11)


def kernel(rays, centers, rays_encoding, near, far, xy, yz, zx,
           xy_color, yz_color, zx_color, mlp_weights, mlp_biases,
           weight_opacity, bias_opacity, weight_color, bias_color, bg_color,
           interpret=False):
    sgrid = (jnp.arange(_S, dtype=jnp.float32) + 0.5) / _S
    t = near[None, :] + (far - near)[None, :] * sgrid[:, None]
    pts = centers[None, :, :] + t[..., None] * rays[None, :, :]
    px, py, pz = pts[..., 0], pts[..., 1], pts[..., 2]

    f = (_bilerp_sr(xy, px, py) + _bilerp_sr(yz, py, pz)
         + _bilerp_sr(zx, pz, px))
    fc = (_bilerp_sr(xy_color, px, py) + _bilerp_sr(yz_color, py, pz)
          + _bilerp_sr(zx_color, pz, px))
    x = jnp.concatenate([f, fc], axis=-1).reshape(_S * _R, 64)

    zero = jnp.zeros((_C, _C), jnp.float32)
    wb = jnp.concatenate([
        jnp.concatenate([_GAIN * mlp_weights[0].T, zero], axis=1),
        jnp.concatenate([zero, _GAIN * jnp.eye(_C, dtype=jnp.float32)], axis=1),
    ], axis=0)

    out = _render(
        x, rays_encoding.T, jnp.stack([near, far]), wb,
        mlp_weights[1].T, mlp_biases[0][:, None], mlp_biases[1][:, None],
        weight_opacity[:, None], bias_opacity.reshape(1, 1),
        weight_color.T, bias_color[:, None], bg_color[:, None],
        interpret=interpret)

    feature_render = out[0:3, :].T
    mask = out[3, :]
    ray_length = out[4, :]
    return feature_render, mask, ray_length


# speedup vs baseline: 115.9102x; 115.9102x over previous
import functools

import jax
import jax.numpy as jnp
from jax import lax
from jax.experimental import pallas as pl
from jax.experimental.pallas import tpu as pltpu
from jax.experimental.pallas import tpu_sc as plsc

_R = 8192
_S = 64
_C = 32
_P = _R * _S
_GAIN = 1.0
_NW = 32
_PW = _P // _NW
_CH = 128


_GDN = lax.GatherDimensionNumbers(offset_dims=(), collapsed_slice_dims=(0,),
                                  start_index_map=(0,))


def _lane_bcast(v, jv):
    return lax.gather(v, jv[:, None], _GDN, (1,),
                      mode=lax.GatherScatterMode.PROMISE_IN_BOUNDS)


def _sc_gather(t_xy, t_yz, t_zx, idx_all, wt_all):
    mesh = plsc.VectorSubcoreMesh(core_axis_name="c", subcore_axis_name="s")

    @functools.partial(
        pl.kernel,
        mesh=mesh,
        out_type=jax.ShapeDtypeStruct((_P, 64), jnp.float32),
        scratch_types=[
            pltpu.VMEM((6, _CH), jnp.int32),
            pltpu.VMEM((6, _CH), jnp.float32),
            *[pltpu.VMEM((_CH, 128), jnp.float32) for _ in range(6)],
            pltpu.VMEM((_CH, 64), jnp.float32),
            pltpu.SemaphoreType.DMA,
        ],
    )
    def k(txy, tyz, tzx, idx_hbm, wt_hbm, out_hbm,
          idx_s, wt_s, g0, g1, g2, g3, g4, g5,
          out_p, sem):
        tabs = (txy, tyz, tzx)
        gbufs = (g0, g1, g2, g3, g4, g5)
        wid = lax.axis_index("s") * 2 + lax.axis_index("c")

        def blk_body(b, carry):
            base = wid * _PW + b * _CH
            pltpu.sync_copy(idx_hbm.at[:, pl.ds(base, _CH)], idx_s)
            pltpu.sync_copy(wt_hbm.at[:, pl.ds(base, _CH)], wt_s)
            copies = []
            for t in range(3):
                for q in range(2):
                    copies.append(pltpu.async_copy(
                        tabs[t].at[idx_s.at[2 * t + q]], gbufs[2 * t + q], sem))
            for cp in copies:
                cp.wait()
            for g16 in range(_CH // 16):
                ws = []
                for t in range(3):
                    tx = wt_s[2 * t, pl.ds(g16 * 16, 16)]
                    ty = wt_s[2 * t + 1, pl.ds(g16 * 16, 16)]
                    ws += [(1.0 - tx) * (1.0 - ty), tx * (1.0 - ty),
                           (1.0 - tx) * ty, tx * ty]

                def pt_body(jj, c2, g16=g16, ws=ws):
                    j = g16 * 16 + jj
                    jv = jnp.zeros((16,), jnp.int32) + jj
                    wb = [_lane_bcast(w, jv) for w in ws]
                    for g in range(4):
                        lo = pl.ds(16 * g, 16)
                        hi = pl.ds(64 + 16 * g, 16)
                        acc = (wb[0] * gbufs[0][j, lo]
                               + wb[1] * gbufs[0][j, hi]
                               + wb[2] * gbufs[1][j, lo]
                               + wb[3] * gbufs[1][j, hi])
                        for t in range(1, 3):
                            acc = (acc + wb[4 * t] * gbufs[2 * t][j, lo]
                                   + wb[4 * t + 1] * gbufs[2 * t][j, hi]
                                   + wb[4 * t + 2] * gbufs[2 * t + 1][j, lo]
                                   + wb[4 * t + 3] * gbufs[2 * t + 1][j, hi])
                        out_p[j, lo] = acc
                    return c2

                lax.fori_loop(0, 16, pt_body, 0)
            pltpu.sync_copy(out_p, out_hbm.at[pl.ds(base, _CH)])
            return carry

        lax.fori_loop(0, _PW // _CH, blk_body, 0)

    return k(t_xy, t_yz, t_zx, idx_all, wt_all)


def _softplus(x):
    return jnp.maximum(x, 0.0) + jnp.log1p(jnp.exp(-jnp.abs(x)))


def _tc_body(x_ref, enc_ref, nf_ref, wb_ref, w1t_ref, b0_ref, b1_ref,
             wop_ref, bop_ref, wct_ref, bc_ref, bg_ref,
             out_ref, nlt_ref, awhc_ref, aw_ref, awt_ref):
    s = pl.program_id(0)

    @pl.when(s == 0)
    def _init():
        nlt_ref[...] = jnp.zeros_like(nlt_ref)
        awhc_ref[...] = jnp.zeros_like(awhc_ref)
        aw_ref[...] = jnp.zeros_like(aw_ref)
        awt_ref[...] = jnp.zeros_like(awt_ref)

    x = x_ref[...]
    z = lax.dot_general(wb_ref[...], x, (((1,), (1,)), ((), ())),
                        preferred_element_type=jnp.float32)
    h = _softplus(z[0:_C, :] + b0_ref[...])
    h = _softplus(jnp.dot(w1t_ref[...], h,
                          preferred_element_type=jnp.float32) + b1_ref[...])
    sig_pre = jnp.sum(h * wop_ref[...], axis=0, keepdims=True) + bop_ref[...]
    sigma = _softplus(sig_pre)

    near = nf_ref[0:1, :]
    far = nf_ref[1:2, :]
    sf = (s.astype(jnp.float32) + 0.5) * (1.0 / _S)
    t = near + (far - near) * sf
    delta = (far - near) * (1.0 / _S)
    sd = sigma * delta

    nlt0 = nlt_ref[...]
    wgt = jnp.exp(-nlt0) * (1.0 - jnp.exp(-sd))
    nlt_ref[...] = nlt0 + sd

    hc = _softplus(h + z[_C:2 * _C, :] + enc_ref[...])
    awhc_ref[...] += wgt * hc
    aw_ref[...] += wgt
    awt_ref[...] += wgt * t

    @pl.when(s == _S - 1)
    def _fin():
        mask = 1.0 - jnp.exp(-nlt_ref[...])
        cf = (jnp.dot(wct_ref[...], awhc_ref[...],
                      preferred_element_type=jnp.float32)
              + bc_ref[...] * aw_ref[...])
        fr = cf[0:3, :] + (1.0 - mask) * bg_ref[...]
        out_ref[...] = jnp.concatenate(
            [fr, mask, awt_ref[...], jnp.zeros((3, _R), jnp.float32)], axis=0)


def _render(x, enc_t, nf, wb, w1t, b0, b1, wop, bop, wct, bc, bg):
    full = lambda shape: pl.BlockSpec(shape, lambda s: (0, 0))
    return pl.pallas_call(
        _tc_body,
        grid=(_S,),
        in_specs=[
            pl.BlockSpec((_R, 64), lambda s: (s, 0)),
            full((_C, _R)), full((2, _R)), full((64, 64)), full((_C, _C)),
            full((_C, 1)), full((_C, 1)), full((_C, 1)), full((1, 1)),
            full((16, _C)), full((16, 1)), full((3, 1)),
        ],
        out_specs=full((8, _R)),
        out_shape=jax.ShapeDtypeStruct((8, _R), jnp.float32),
        scratch_shapes=[
            pltpu.VMEM((1, _R), jnp.float32),
            pltpu.VMEM((_C, _R), jnp.float32),
            pltpu.VMEM((1, _R), jnp.float32),
            pltpu.VMEM((1, _R), jnp.float32),
        ],
        compiler_params=pltpu.CompilerParams(
            dimension_semantics=("arbitrary",)),
    )(x, enc_t, nf, wb, w1t, b0, b1, wop, bop, wct, bc, bg)


def _corner_idx(u, v):
    x = jnp.clip((u + 1.0) * (0.5 * 255.0), 0.0, 255.0)
    y = jnp.clip((v + 1.0) * (0.5 * 255.0), 0.0, 255.0)
    x0 = jnp.floor(x)
    y0 = jnp.floor(y)
    tx = x - x0
    ty = y - y0
    x0i = x0.astype(jnp.int32)
    y0i = y0.astype(jnp.int32)
    y1i = jnp.minimum(y0i + 1, 255)
    i00 = y0i * 256 + x0i
    i10 = y1i * 256 + x0i
    return (i00, i10), tx, ty


def _pack_table(feat, color):
    base = jnp.concatenate([feat, color], axis=0).transpose(1, 2, 0).reshape(
        256 * 256, 64)
    shifted = jnp.concatenate([base[1:], base[-1:]], axis=0)
    return jnp.concatenate([base, shifted], axis=1)


def kernel(rays, centers, rays_encoding, near, far, xy, yz, zx,
           xy_color, yz_color, zx_color, mlp_weights, mlp_biases,
           weight_opacity, bias_opacity, weight_color, bias_color, bg_color):
    sgrid = (jnp.arange(_S, dtype=jnp.float32) + 0.5) / _S
    t = near[None, :] + (far - near)[None, :] * sgrid[:, None]
    pts = centers[None, :, :] + t[..., None] * rays[None, :, :]
    px = pts[..., 0].reshape(_P)
    py = pts[..., 1].reshape(_P)
    pz = pts[..., 2].reshape(_P)

    idx_rows = []
    wt_rows = []
    for u, v in ((px, py), (py, pz), (pz, px)):
        idx2, tx, ty = _corner_idx(u, v)
        idx_rows += list(idx2)
        wt_rows += [tx, ty]
    idx_all = jnp.stack(idx_rows)
    wt_all = jnp.stack(wt_rows)

    t_xy = _pack_table(xy, xy_color)
    t_yz = _pack_table(yz, yz_color)
    t_zx = _pack_table(zx, zx_color)

    x = _sc_gather(t_xy, t_yz, t_zx, idx_all, wt_all)

    zero = jnp.zeros((_C, _C), jnp.float32)
    wb = jnp.concatenate([
        jnp.concatenate([_GAIN * mlp_weights[0].T, zero], axis=1),
        jnp.concatenate([zero, _GAIN * jnp.eye(_C, dtype=jnp.float32)], axis=1),
    ], axis=0)

    out = _render(
        x, rays_encoding.T, jnp.stack([near, far]), wb,
        mlp_weights[1].T, mlp_biases[0][:, None], mlp_biases[1][:, None],
        weight_opacity[:, None], bias_opacity.reshape(1, 1),
        weight_color.T, bias_color[:, None], bg_color[:, None])

    feature_render = out[0:3, :].T
    mask = out[3, :]
    ray_length = out[4, :]
    return feature_render, mask, ray_length
